# Initial kernel scaffold; baseline (speedup 1.0000x reference)
import jax
import jax.numpy as jnp
from jax.experimental import pallas as pl


def kernel(x, x_nb):
    raise NotImplementedError("write your pallas kernel here")


# speedup vs baseline: 1.7319x; 1.7319x over previous
import functools

import jax
import jax.numpy as jnp
from jax import lax
from jax.experimental import pallas as pl
from jax.experimental.pallas import tpu as pltpu
from jax.experimental.pallas import tpu_sc as plsc

_D = 128
_M = 32
_N = 10000
_ITERS = 3

_TOT = _N * _M
_CH = 80


def _sc_gather(x, idx3):
    info = plsc.get_sparse_core_info()
    nw = info.num_cores * info.num_subcores
    per_w = _TOT // nw
    n_ch = per_w // _CH

    mesh = plsc.VectorSubcoreMesh(core_axis_name="c", subcore_axis_name="s")

    @functools.partial(
        pl.kernel,
        mesh=mesh,
        out_type=jax.ShapeDtypeStruct((_TOT, _D), jnp.float32),
        scratch_types=[
            pltpu.VMEM((n_ch, _CH), jnp.int32),
            pltpu.VMEM((_CH, _D), jnp.float32),
            pltpu.SemaphoreType.DMA,
        ],
    )
    def k(x_hbm, idx_hbm, z_hbm, idx_v, rows_v, sem):
        wid = lax.axis_index("s") * info.num_cores + lax.axis_index("c")
        chunk0 = wid * n_ch
        pltpu.sync_copy(idx_hbm.at[wid], idx_v)

        def body(j, carry):
            pltpu.async_copy(x_hbm.at[idx_v.at[j]], rows_v, sem).wait()
            base = (chunk0 + j) * _CH
            pltpu.sync_copy(rows_v, z_hbm.at[pl.ds(base, _CH)])
            return carry

        lax.fori_loop(0, n_ch, body, 0)

    return k(x, idx3)


def _routing_body(z_ref, x_ref, o_ref):
    z = z_ref[...]
    xb = x_ref[...]

    zn = jnp.sqrt(jnp.sum(z * z, axis=2, keepdims=True))
    z = z / jnp.maximum(zn, 1e-12)
    xn = jnp.sqrt(jnp.sum(xb * xb, axis=1, keepdims=True))
    xb = xb / jnp.maximum(xn, 1e-12)

    u = jnp.mean(z, axis=1) + xb

    for it in range(1, _ITERS):
        n2 = jnp.sum(u * u, axis=1, keepdims=True)
        nrm = jnp.sqrt(n2)
        u = (n2 / (n2 + 1.0)) * (u / jnp.maximum(nrm, 1e-12))

        p = jnp.sum(z * u[:, None, :], axis=2)
        p = p - jnp.max(p, axis=1, keepdims=True)
        e = jnp.exp(p)
        p = e / jnp.sum(e, axis=1, keepdims=True)
        u = jnp.sum(z * p[:, :, None], axis=1) + xb

    o_ref[...] = u


def _tc_routing(z3, x):
    b = 200
    return pl.pallas_call(
        _routing_body,
        grid=(_N // b,),
        in_specs=[
            pl.BlockSpec((b, _M, _D), lambda i: (i, 0, 0)),
            pl.BlockSpec((b, _D), lambda i: (i, 0)),
        ],
        out_specs=pl.BlockSpec((b, _D), lambda i: (i, 0)),
        out_shape=jax.ShapeDtypeStruct((_N, _D), jnp.float32),
    )(z3, x)


def kernel(x, x_nb):
    idx = jnp.where(x_nb == 0, _N - 1, x_nb - 1).astype(jnp.int32)
    idx3 = idx.reshape(32, _TOT // (32 * _CH), _CH)
    z = _sc_gather(x, idx3)
    z3 = z.reshape(_N, _M, _D)
    return _tc_routing(z3, x)


# speedup vs baseline: 2.5754x; 1.4870x over previous
import functools

import jax
import jax.numpy as jnp
from jax import lax
from jax.experimental import pallas as pl
from jax.experimental.pallas import tpu as pltpu
from jax.experimental.pallas import tpu_sc as plsc

_D = 128
_M = 32
_N = 10000
_ITERS = 3

_TOT = _N * _M
_CH = 80


def _sc_gather(x, idx3):
    info = plsc.get_sparse_core_info()
    nw = info.num_cores * info.num_subcores
    per_w = _TOT // nw
    n_ch = per_w // _CH

    mesh = plsc.VectorSubcoreMesh(core_axis_name="c", subcore_axis_name="s")

    @functools.partial(
        pl.kernel,
        mesh=mesh,
        out_type=jax.ShapeDtypeStruct((_TOT, _D), jnp.float32),
        scratch_types=[
            pltpu.VMEM((n_ch, _CH), jnp.int32),
            pltpu.VMEM((_CH, _D), jnp.float32),
            pltpu.SemaphoreType.DMA,
        ],
    )
    def k(x_hbm, idx_hbm, z_hbm, idx_v, rows_v, sem):
        wid = lax.axis_index("s") * info.num_cores + lax.axis_index("c")
        chunk0 = wid * n_ch
        pltpu.sync_copy(idx_hbm.at[wid], idx_v)

        def body(j, carry):
            pltpu.async_copy(x_hbm.at[idx_v.at[j]], rows_v, sem).wait()
            base = (chunk0 + j) * _CH
            pltpu.sync_copy(rows_v, z_hbm.at[pl.ds(base, _CH)])
            return carry

        lax.fori_loop(0, n_ch, body, 0)

    return k(x, idx3)


_B = 200


def _routing_body(z_ref, x_ref, o_ref):
    b = _B
    zf = z_ref[...]
    xb = x_ref[...]

    ones = jnp.ones((_D, _D), jnp.float32)

    s = jnp.dot(zf * zf, ones)
    zf = zf * lax.rsqrt(jnp.maximum(s, 1e-24))
    sx = jnp.dot(xb * xb, ones)
    xb = xb * lax.rsqrt(jnp.maximum(sx, 1e-24))

    z = zf.reshape(b, _M, _D)

    u = jnp.sum(z, axis=1) * (1.0 / _M) + xb

    for it in range(1, _ITERS):
        n2 = jnp.dot(u * u, ones)
        u = u * (n2 * lax.rsqrt(jnp.maximum(n2, 1e-24)) / (n2 + 1.0))

        t = z * u[:, None, :]
        d = jnp.dot(t.reshape(b * _M, _D), ones).reshape(b, _M, _D)
        e = jnp.exp(d)
        num = jnp.sum(e * z, axis=1)
        den = jnp.sum(e, axis=1)
        u = num / den + xb

    o_ref[...] = u


def _tc_routing(z, x):
    return pl.pallas_call(
        _routing_body,
        grid=(_N // _B,),
        in_specs=[
            pl.BlockSpec((_B * _M, _D), lambda i: (i, 0)),
            pl.BlockSpec((_B, _D), lambda i: (i, 0)),
        ],
        out_specs=pl.BlockSpec((_B, _D), lambda i: (i, 0)),
        out_shape=jax.ShapeDtypeStruct((_N, _D), jnp.float32),
    )(z, x)


def kernel(x, x_nb):
    idx = jnp.where(x_nb == 0, _N - 1, x_nb - 1).astype(jnp.int32)
    idx3 = idx.reshape(32, _TOT // (32 * _CH), _CH)
    z = _sc_gather(x, idx3)
    return _tc_routing(z, x)
